# Initial kernel scaffold; baseline (speedup 1.0000x reference)
import functools

import jax
import jax.numpy as jnp
from jax import lax
from jax.experimental import pallas as pl
from jax.experimental.pallas import tpu as pltpu
from jax.experimental.pallas import tpu_sc as plsc

THRESHOLD = 1000000
EMBED_DIM = 64
CHUNK = 128
LANES = 16


def _sc_embed(tokens_flat, base_table, ext_table, *, n_tokens, ext_rows):
  info = plsc.get_sparse_core_info()
  nc, ns = info.num_cores, info.num_subcores
  nw = nc * ns
  assert n_tokens % (nw * CHUNK) == 0
  per_worker = n_tokens // nw
  n_chunks = per_worker // CHUNK

  mesh = plsc.VectorSubcoreMesh(core_axis_name="c", subcore_axis_name="s")

  @functools.partial(
      pl.kernel,
      mesh=mesh,
      out_type=jax.ShapeDtypeStruct((n_tokens, EMBED_DIM), jnp.float32),
      scratch_types=[
          pltpu.VMEM((ext_rows, EMBED_DIM), jnp.float32),
          pltpu.VMEM((CHUNK,), jnp.int32),
          pltpu.VMEM((CHUNK,), jnp.int32),
          pltpu.VMEM((CHUNK + LANES,), jnp.int32),
          pltpu.VMEM((CHUNK + LANES,), jnp.int32),
          pltpu.VMEM((CHUNK, EMBED_DIM), jnp.float32),
          pltpu.SMEM((CHUNK + LANES,), jnp.int32),
          pltpu.SMEM((CHUNK + LANES,), jnp.int32),
          pltpu.SemaphoreType.DMA,
      ],
  )
  def k(tok_hbm, base_hbm, ext_hbm, out_hbm,
        ext_v, tok_v, bidx_v, eidx_v, pos_v, rows_v, eidx_s, pos_s, sem):
    wid = lax.axis_index("s") * nc + lax.axis_index("c")
    w_base = wid * per_worker

    pltpu.sync_copy(ext_hbm, ext_v)

    def chunk_body(g, _):
      base = w_base + g * CHUNK
      pltpu.sync_copy(tok_hbm.at[pl.ds(base, CHUNK)], tok_v)

      def group_body(j, off):
        tok = tok_v[pl.ds(j * LANES, LANES)]
        m = tok >= THRESHOLD
        bidx = jnp.where(m, 0, tok)
        bidx_v[pl.ds(j * LANES, LANES)] = bidx
        cnt = jnp.sum(m.astype(jnp.int32))

        @pl.when(cnt > 0)
        def _():
          eidx = tok - THRESHOLD
          pos = lax.iota(jnp.int32, LANES) + j * LANES
          plsc.store_compressed(eidx_v.at[pl.ds(off, LANES)], eidx, m)
          plsc.store_compressed(pos_v.at[pl.ds(off, LANES)], pos, m)

        return off + cnt

      total = lax.fori_loop(0, CHUNK // LANES, group_body, 0)

      pltpu.async_copy(base_hbm.at[bidx_v], rows_v, sem).wait()

      @pl.when(total > 0)
      def _():
        pltpu.sync_copy(eidx_v, eidx_s)
        pltpu.sync_copy(pos_v, pos_s)

        def fix(i, _):
          e = eidx_s[i]
          p = pos_s[i]
          for c in range(EMBED_DIM // LANES):
            rows_v[p, pl.ds(c * LANES, LANES)] = ext_v[e, pl.ds(c * LANES, LANES)]
          return 0

        lax.fori_loop(0, total, fix, 0)

      pltpu.sync_copy(rows_v, out_hbm.at[pl.ds(base, CHUNK)])
      return 0

    lax.fori_loop(0, n_chunks, chunk_body, 0)

  return k(tokens_flat, base_table, ext_table)


def kernel(input_tokens, base_table, ext_table):
  b, s = input_tokens.shape
  n_tokens = b * s
  out = _sc_embed(
      input_tokens.reshape(n_tokens),
      base_table,
      ext_table,
      n_tokens=n_tokens,
      ext_rows=ext_table.shape[0],
  )
  return out.reshape(b, s, EMBED_DIM)


# speedup vs baseline: 2.2993x; 2.2993x over previous
import functools

import jax
import jax.numpy as jnp
from jax import lax
from jax.experimental import pallas as pl
from jax.experimental.pallas import tpu as pltpu
from jax.experimental.pallas import tpu_sc as plsc

THRESHOLD = 1000000
EMBED_DIM = 64
CHUNK = 128
LANES = 16


def _sc_embed(tokens_flat, base_table, ext_table, *, n_tokens, ext_rows):
  info = plsc.get_sparse_core_info()
  nc, ns = info.num_cores, info.num_subcores
  nw = nc * ns
  assert n_tokens % (nw * CHUNK) == 0
  per_worker = n_tokens // nw
  n_chunks = per_worker // CHUNK

  mesh = plsc.VectorSubcoreMesh(core_axis_name="c", subcore_axis_name="s")

  @functools.partial(
      pl.kernel,
      mesh=mesh,
      compiler_params=pltpu.CompilerParams(
          use_tc_tiling_on_sc=False, needs_layout_passes=False),
      out_type=jax.ShapeDtypeStruct((n_tokens, EMBED_DIM), jnp.float32),
      scratch_types=[
          pltpu.VMEM((ext_rows, EMBED_DIM), jnp.float32),
          pltpu.VMEM((CHUNK,), jnp.int32),
          pltpu.VMEM((CHUNK,), jnp.int32),
          pltpu.VMEM((CHUNK + LANES,), jnp.int32),
          pltpu.VMEM((CHUNK + LANES,), jnp.int32),
          pltpu.VMEM((CHUNK, EMBED_DIM), jnp.float32),
          pltpu.SemaphoreType.DMA,
      ],
  )
  def k(tok_hbm, base_hbm, ext_hbm, out_hbm,
        ext_v, tok_v, bidx_v, eidx_v, pos_v, rows_v, sem):
    wid = lax.axis_index("s") * nc + lax.axis_index("c")
    w_base = wid * per_worker

    pltpu.sync_copy(ext_hbm, ext_v)

    ones = jnp.full((LANES,), 1, jnp.int32)
    zeros = jnp.full((LANES,), 0, jnp.int32)

    def chunk_body(g, _):
      base = w_base + g * CHUNK
      pltpu.sync_copy(tok_hbm.at[pl.ds(base, CHUNK)], tok_v)

      def group_body(j, off):
        tok = tok_v[pl.ds(j * LANES, LANES)]
        m = tok >= THRESHOLD
        bidx = jnp.where(m, zeros, tok)
        bidx_v[pl.ds(j * LANES, LANES)] = bidx
        cnt = jnp.sum(jnp.where(m, ones, zeros))

        @pl.when(cnt > 0)
        def _():
          eidx = tok - THRESHOLD
          pos = lax.iota(jnp.int32, LANES) + j * LANES
          plsc.store_compressed(eidx_v.at[pl.ds(off, LANES)], eidx, mask=m)
          plsc.store_compressed(pos_v.at[pl.ds(off, LANES)], pos, mask=m)

        return off + cnt

      total = lax.fori_loop(0, CHUNK // LANES, group_body, 0)

      pltpu.async_copy(base_hbm.at[bidx_v], rows_v, sem).wait()

      @pl.when(total > 0)
      def _():
        lane = lax.iota(jnp.int32, LANES)

        def fix(b, _):
          e = eidx_v[pl.ds(b * LANES, LANES)]
          p = pos_v[pl.ds(b * LANES, LANES)]
          valid = (lane + b * LANES) < total
          e = jnp.where(valid, e, zeros)
          p = jnp.where(valid, p, zeros)
          for c in range(EMBED_DIM):
            col = jnp.full((LANES,), c, jnp.int32)
            vals = plsc.load_gather(ext_v, [e, col])
            plsc.store_scatter(rows_v, [p, col], vals, mask=valid)
          return 0

        lax.fori_loop(0, (total + LANES - 1) // LANES, fix, 0)

      pltpu.sync_copy(rows_v, out_hbm.at[pl.ds(base, CHUNK)])
      return 0

    lax.fori_loop(0, n_chunks, chunk_body, 0)

  return k(tokens_flat, base_table, ext_table)


def kernel(input_tokens, base_table, ext_table):
  b, s = input_tokens.shape
  n_tokens = b * s
  out = _sc_embed(
      input_tokens.reshape(n_tokens),
      base_table,
      ext_table,
      n_tokens=n_tokens,
      ext_rows=ext_table.shape[0],
  )
  return out.reshape(b, s, EMBED_DIM)


# speedup vs baseline: 2.8218x; 1.2273x over previous
import functools

import jax
import jax.numpy as jnp
from jax import lax
from jax.experimental import pallas as pl
from jax.experimental.pallas import tpu as pltpu
from jax.experimental.pallas import tpu_sc as plsc

THRESHOLD = 1000000
EMBED_DIM = 64
CHUNK = 128
LANES = 16
NBUF = 4


def _sc_embed(tokens_flat, base_table, ext_table, *, n_tokens, ext_rows):
  info = plsc.get_sparse_core_info()
  nc, ns = info.num_cores, info.num_subcores
  nw = nc * ns
  assert n_tokens % (nw * CHUNK * NBUF) == 0
  per_worker = n_tokens // nw
  n_chunks = per_worker // CHUNK

  mesh = plsc.VectorSubcoreMesh(core_axis_name="c", subcore_axis_name="s")

  @functools.partial(
      pl.kernel,
      mesh=mesh,
      compiler_params=pltpu.CompilerParams(
          use_tc_tiling_on_sc=False, needs_layout_passes=False),
      out_type=jax.ShapeDtypeStruct((n_tokens, EMBED_DIM), jnp.float32),
      scratch_types=[
          pltpu.VMEM((ext_rows, EMBED_DIM), jnp.float32),
          pltpu.VMEM((NBUF, CHUNK), jnp.int32),
          pltpu.VMEM((NBUF, CHUNK), jnp.int32),
          pltpu.VMEM((NBUF, CHUNK + LANES), jnp.int32),
          pltpu.VMEM((NBUF, CHUNK + LANES), jnp.int32),
          pltpu.VMEM((NBUF, CHUNK, EMBED_DIM), jnp.float32),
          [pltpu.SemaphoreType.DMA] * NBUF,
          [pltpu.SemaphoreType.DMA] * NBUF,
          [pltpu.SemaphoreType.DMA] * NBUF,
      ],
  )
  def k(tok_hbm, base_hbm, ext_hbm, out_hbm,
        ext_v, tok_v, bidx_v, eidx_v, pos_v, rows_v,
        tok_sems, gat_sems, scat_sems):
    wid = lax.axis_index("s") * nc + lax.axis_index("c")
    w_base = wid * per_worker

    pltpu.sync_copy(ext_hbm, ext_v)

    ones = jnp.full((LANES,), 1, jnp.int32)
    zeros = jnp.full((LANES,), 0, jnp.int32)
    lane = lax.iota(jnp.int32, LANES)

    def tok_slice(g):
      return tok_hbm.at[pl.ds(w_base + g * CHUNK, CHUNK)]

    def out_slice(g):
      return out_hbm.at[pl.ds(w_base + g * CHUNK, CHUNK)]

    for b in range(NBUF):
      pltpu.async_copy(tok_slice(b), tok_v.at[b], tok_sems[b])

    def round_body(i, _):
      gg = i * NBUF
      totals = []
      gathers = []

      for b in range(NBUF):
        g = gg + b
        pltpu.make_async_copy(tok_slice(g), tok_v.at[b], tok_sems[b]).wait()

        def group_body(j, off, b=b):
          tok = tok_v[b, pl.ds(j * LANES, LANES)]
          m = tok >= THRESHOLD
          bidx = jnp.where(m, zeros, tok)
          bidx_v[b, pl.ds(j * LANES, LANES)] = bidx
          cnt = jnp.sum(jnp.where(m, ones, zeros))

          @pl.when(cnt > 0)
          def _():
            eidx = tok - THRESHOLD
            pos = lax.iota(jnp.int32, LANES) + j * LANES
            plsc.store_compressed(eidx_v.at[b, pl.ds(off, LANES)], eidx,
                                  mask=m)
            plsc.store_compressed(pos_v.at[b, pl.ds(off, LANES)], pos,
                                  mask=m)

          return off + cnt

        totals.append(lax.fori_loop(0, CHUNK // LANES, group_body, 0))

        @pl.when(i > 0)
        def _(b=b, g=g):
          pltpu.make_async_copy(
              rows_v.at[b], out_slice(g - NBUF), scat_sems[b]).wait()

        gathers.append(
            pltpu.async_copy(base_hbm.at[bidx_v.at[b]], rows_v.at[b],
                             gat_sems[b]))

      for b in range(NBUF):
        nxt = gg + NBUF + b

        @pl.when(nxt < n_chunks)
        def _(b=b, nxt=nxt):
          pltpu.async_copy(tok_slice(nxt), tok_v.at[b], tok_sems[b])

      for b in range(NBUF):
        g = gg + b
        total = totals[b]
        gathers[b].wait()

        @pl.when(total > 0)
        def _(b=b, total=total):
          def fix(f, _):
            e = eidx_v[b, pl.ds(f * LANES, LANES)]
            p = pos_v[b, pl.ds(f * LANES, LANES)]
            valid = (lane + f * LANES) < total
            e = jnp.where(valid, e, zeros)
            p = jnp.where(valid, p, zeros)
            for c in range(EMBED_DIM):
              col = jnp.full((LANES,), c, jnp.int32)
              vals = plsc.load_gather(ext_v, [e, col])
              plsc.store_scatter(rows_v.at[b], [p, col], vals, mask=valid)
            return 0

          lax.fori_loop(0, (total + LANES - 1) // LANES, fix, 0)

        pltpu.async_copy(rows_v.at[b], out_slice(g), scat_sems[b])

      return 0

    lax.fori_loop(0, n_chunks // NBUF, round_body, 0)

    for b in range(NBUF):
      pltpu.make_async_copy(
          rows_v.at[b], out_slice(n_chunks - NBUF + b), scat_sems[b]).wait()

  return k(tokens_flat, base_table, ext_table)


def kernel(input_tokens, base_table, ext_table):
  b, s = input_tokens.shape
  n_tokens = b * s
  out = _sc_embed(
      input_tokens.reshape(n_tokens),
      base_table,
      ext_table,
      n_tokens=n_tokens,
      ext_rows=ext_table.shape[0],
  )
  return out.reshape(b, s, EMBED_DIM)
